# Initial kernel scaffold; baseline (speedup 1.0000x reference)
import jax
import jax.numpy as jnp
from jax.experimental import pallas as pl


def kernel(inputs, codebook):
    raise NotImplementedError("write your pallas kernel here")


# speedup vs baseline: 1.3809x; 1.3809x over previous
import jax
import jax.numpy as jnp
from jax import lax
from jax.experimental import pallas as pl
from jax.experimental.pallas import tpu as pltpu
from jax.experimental.pallas import tpu_sc as plsc

N_TOK = 16384
N_CODE = 1024
DIM = 64
BLK = 2048
GRID = N_TOK // BLK
LOSS_SCALE = 1.25 / (N_TOK * DIM)

SC_CORES = 2
SC_SUBCORES = 16
SC_WORKERS = SC_CORES * SC_SUBCORES
B_PER_W = N_TOK // SC_WORKERS


def _dist_argmin_body(x_ref, cb_ref, idx_ref, loss_ref):
    i = pl.program_id(0)
    x = x_ref[...]
    cb = cb_ref[...]
    xn = jnp.sum(x * x, axis=1, keepdims=True)
    cn = jnp.sum(cb * cb, axis=1)
    xc = lax.dot_general(x, cb, (((1,), (1,)), ((), ())),
                         preferred_element_type=jnp.float32)
    dist = xn - 2.0 * xc + cn[None, :]
    minval = jnp.min(dist, axis=1, keepdims=True)
    iota = lax.broadcasted_iota(jnp.int32, (BLK, N_CODE), 1)
    idx = jnp.min(jnp.where(dist == minval, iota, N_CODE), axis=1)
    idx_ref[...] = idx

    @pl.when(i == 0)
    def _init():
        loss_ref[0, 0] = 0.0

    loss_ref[0, 0] += jnp.sum(minval)

    @pl.when(i == GRID - 1)
    def _finish():
        loss_ref[0, 0] = loss_ref[0, 0] * LOSS_SCALE


_dist_argmin = pl.pallas_call(
    _dist_argmin_body,
    grid=(GRID,),
    in_specs=[
        pl.BlockSpec((BLK, DIM), lambda i: (i, 0)),
        pl.BlockSpec((N_CODE, DIM), lambda i: (0, 0)),
    ],
    out_specs=[
        pl.BlockSpec((BLK,), lambda i: (i,)),
        pl.BlockSpec((1, 1), lambda i: (0, 0), memory_space=pltpu.SMEM),
    ],
    out_shape=[
        jax.ShapeDtypeStruct((N_TOK,), jnp.int32),
        jax.ShapeDtypeStruct((1, 1), jnp.float32),
    ],
)


def _gather_body(cb_hbm, idx_hbm, out_hbm, idx_v, rows_v, sem):
    wid = lax.axis_index("s") * SC_CORES + lax.axis_index("c")
    base = wid * B_PER_W
    pltpu.sync_copy(idx_hbm.at[pl.ds(base, B_PER_W)], idx_v)
    pltpu.async_copy(cb_hbm.at[idx_v], rows_v, sem).wait()
    pltpu.sync_copy(rows_v, out_hbm.at[pl.ds(base, B_PER_W)])


def _make_gather():
    return pl.kernel(
        _gather_body,
        mesh=plsc.VectorSubcoreMesh(core_axis_name="c", subcore_axis_name="s"),
        out_type=jax.ShapeDtypeStruct((N_TOK, DIM), jnp.float32),
        scratch_types=[
            pltpu.VMEM((B_PER_W,), jnp.int32),
            pltpu.VMEM((B_PER_W, DIM), jnp.float32),
            pltpu.SemaphoreType.DMA,
        ],
        compiler_params=pltpu.CompilerParams(use_tc_tiling_on_sc=False),
    )


def kernel(inputs, codebook):
    idx, loss = _dist_argmin(inputs, codebook)
    quantized = _make_gather()(codebook, idx)
    return quantized, loss[0, 0], idx


# speedup vs baseline: 2.0693x; 1.4986x over previous
import jax
import jax.numpy as jnp
from jax import lax
from jax.experimental import pallas as pl
from jax.experimental.pallas import tpu as pltpu
from jax.experimental.pallas import tpu_sc as plsc

N_TOK = 16384
N_CODE = 1024
DIM = 64
BLK = 4096
NCHUNK = 1
TOK_CHUNK = N_TOK // NCHUNK
LANES = 128
NCH = N_CODE // LANES
LOSS_SCALE = 1.25 / (N_TOK * DIM)

SC_CORES = 2
SC_SUBCORES = 16
SC_WORKERS = SC_CORES * SC_SUBCORES
B_PER_W = TOK_CHUNK // SC_WORKERS
PAD_DIM = 128


def _dist_argmin_body(xt_ref, cbt_ref, idx_ref, loss_ref):
    i = pl.program_id(0)
    xt = xt_ref[...]
    cbt = cbt_ref[...]
    xn = jnp.sum(xt * xt, axis=0)
    cn = jnp.sum(cbt * cbt, axis=0)[:, None]
    m2xt = xt * -2.0
    xc2 = lax.dot_general(cbt, m2xt, (((0,), (0,)), ((), ())),
                          preferred_element_type=jnp.float32)
    xnr = xn[None, :]
    best_v = (xnr + xc2[0:LANES, :]) + cn[0:LANES]
    best_c = jnp.zeros((LANES, BLK), jnp.float32)
    for k in range(1, NCH):
        v = (xnr + xc2[k * LANES:(k + 1) * LANES, :]) + cn[k * LANES:(k + 1) * LANES]
        lt = v < best_v
        best_v = jnp.where(lt, v, best_v)
        best_c = jnp.where(lt, jnp.full((LANES, BLK), float(k), jnp.float32),
                           best_c)
    row = lax.broadcasted_iota(jnp.int32, (LANES, BLK), 0).astype(jnp.float32)
    best_j = best_c * float(LANES) + row
    minval = jnp.min(best_v, axis=0, keepdims=True)
    masked = jnp.where(best_v == minval, best_j, float(N_CODE))
    idxf = jnp.min(masked, axis=0)
    idx_ref[...] = idxf.astype(jnp.int32)

    @pl.when(i == 0)
    def _init():
        loss_ref[0, 0] = 0.0

    loss_ref[0, 0] += jnp.sum(minval)


def _make_dist_argmin(chunk):
    blk_off = chunk * (TOK_CHUNK // BLK)
    return pl.pallas_call(
        _dist_argmin_body,
        grid=(TOK_CHUNK // BLK,),
        in_specs=[
            pl.BlockSpec((DIM, BLK), lambda i: (0, i + blk_off)),
            pl.BlockSpec((DIM, N_CODE), lambda i: (0, 0)),
        ],
        out_specs=[
            pl.BlockSpec((BLK,), lambda i: (i,)),
            pl.BlockSpec((1, 1), lambda i: (0, 0), memory_space=pltpu.SMEM),
        ],
        out_shape=[
            jax.ShapeDtypeStruct((TOK_CHUNK,), jnp.int32),
            jax.ShapeDtypeStruct((1, 1), jnp.float32),
        ],
    )


def _gather_body(cb_hbm, idx_hbm, out_hbm, idx_v, rows_v, sem):
    wid = lax.axis_index("s") * SC_CORES + lax.axis_index("c")
    base = wid * B_PER_W
    pltpu.sync_copy(idx_hbm.at[pl.ds(base, B_PER_W)], idx_v)
    pltpu.async_copy(cb_hbm.at[idx_v], rows_v, sem).wait()
    pltpu.sync_copy(rows_v, out_hbm.at[pl.ds(base, B_PER_W), pl.ds(0, DIM)])


def _make_gather():
    return pl.kernel(
        _gather_body,
        mesh=plsc.VectorSubcoreMesh(core_axis_name="c", subcore_axis_name="s"),
        out_type=jax.ShapeDtypeStruct((TOK_CHUNK, PAD_DIM), jnp.float32),
        scratch_types=[
            pltpu.VMEM((B_PER_W,), jnp.int32),
            pltpu.VMEM((B_PER_W, DIM), jnp.float32),
            pltpu.SemaphoreType.DMA,
        ],
        compiler_params=pltpu.CompilerParams(use_tc_tiling_on_sc=False),
    )


TR_BLK = 4096


def _tr_body_first(g_ref, qt_ref):
    qt_ref[...] = g_ref[:, 0:DIM].T


def _tr_body_acc(g_ref, qt_in_ref, qt_ref):
    del qt_in_ref
    qt_ref[...] = g_ref[:, 0:DIM].T


def _make_transpose(chunk):
    blk_off = chunk * (TOK_CHUNK // TR_BLK)
    g_spec = pl.BlockSpec((TR_BLK, PAD_DIM), lambda i: (i, 0))
    out_spec = pl.BlockSpec((DIM, TR_BLK), lambda i: (0, i + blk_off))
    out_shape = jax.ShapeDtypeStruct((DIM, N_TOK), jnp.float32)
    if chunk == 0:
        return pl.pallas_call(
            _tr_body_first,
            grid=(TOK_CHUNK // TR_BLK,),
            in_specs=[g_spec],
            out_specs=out_spec,
            out_shape=out_shape,
        )
    return pl.pallas_call(
        _tr_body_acc,
        grid=(TOK_CHUNK // TR_BLK,),
        in_specs=[g_spec, pl.BlockSpec(memory_space=pl.ANY)],
        out_specs=out_spec,
        out_shape=out_shape,
        input_output_aliases={1: 0},
    )


def kernel(inputs, codebook):
    xt = inputs.T
    cbt = codebook.T
    idxs, gathers = [], []
    loss_sum = None
    for c in range(NCHUNK):
        idx_c, loss_c = _make_dist_argmin(c)(xt, cbt)
        g_c = _make_gather()(codebook, idx_c)
        idxs.append(idx_c)
        gathers.append(g_c)
        s = loss_c[0, 0]
        loss_sum = s if loss_sum is None else loss_sum + s
    qt = _make_transpose(0)(gathers[0])
    for c in range(1, NCHUNK):
        qt = _make_transpose(c)(gathers[c], qt)
    if NCHUNK == 1:
        idx = idxs[0]
    else:
        idx = jnp.zeros((N_TOK,), jnp.int32)
        for c in range(NCHUNK):
            idx = lax.dynamic_update_slice(idx, idxs[c], (c * TOK_CHUNK,))
    return qt.T, loss_sum * LOSS_SCALE, idx


# speedup vs baseline: 2.1181x; 1.0236x over previous
import jax
import jax.numpy as jnp
from jax import lax
from jax.experimental import pallas as pl
from jax.experimental.pallas import tpu as pltpu
from jax.experimental.pallas import tpu_sc as plsc

N_TOK = 16384
N_CODE = 1024
DIM = 64
BLK = 4096
NCHUNK = 2
TOK_CHUNK = N_TOK // NCHUNK
LANES = 128
NCH = N_CODE // LANES
LOSS_SCALE = 1.25 / (N_TOK * DIM)

SC_CORES = 2
SC_SUBCORES = 16
SC_WORKERS = SC_CORES * SC_SUBCORES
B_PER_W = TOK_CHUNK // SC_WORKERS
PAD_DIM = 128


def _dist_argmin_body(xt_ref, cbt_ref, idx_ref, loss_ref):
    i = pl.program_id(0)
    xt = xt_ref[...]
    cbt = cbt_ref[...]
    xn = jnp.sum(xt * xt, axis=0)
    cn = jnp.sum(cbt * cbt, axis=0)[:, None]
    m2xt = xt * -2.0
    xc2 = lax.dot_general(cbt, m2xt, (((0,), (0,)), ((), ())),
                          preferred_element_type=jnp.float32)
    xnr = xn[None, :]
    best_v = (xnr + xc2[0:LANES, :]) + cn[0:LANES]
    best_c = jnp.zeros((LANES, BLK), jnp.float32)
    for k in range(1, NCH):
        v = (xnr + xc2[k * LANES:(k + 1) * LANES, :]) + cn[k * LANES:(k + 1) * LANES]
        lt = v < best_v
        best_v = jnp.where(lt, v, best_v)
        best_c = jnp.where(lt, jnp.full((LANES, BLK), float(k), jnp.float32),
                           best_c)
    row = lax.broadcasted_iota(jnp.int32, (LANES, BLK), 0).astype(jnp.float32)
    best_j = best_c * float(LANES) + row
    minval = jnp.min(best_v, axis=0, keepdims=True)
    masked = jnp.where(best_v == minval, best_j, float(N_CODE))
    idxf = jnp.min(masked, axis=0)
    idx_ref[...] = idxf.astype(jnp.int32)

    @pl.when(i == 0)
    def _init():
        loss_ref[0, 0] = 0.0

    loss_ref[0, 0] += jnp.sum(minval)


def _make_dist_argmin(chunk):
    blk_off = chunk * (TOK_CHUNK // BLK)
    return pl.pallas_call(
        _dist_argmin_body,
        grid=(TOK_CHUNK // BLK,),
        in_specs=[
            pl.BlockSpec((DIM, BLK), lambda i: (0, i + blk_off)),
            pl.BlockSpec((DIM, N_CODE), lambda i: (0, 0)),
        ],
        out_specs=[
            pl.BlockSpec((BLK,), lambda i: (i,)),
            pl.BlockSpec((1, 1), lambda i: (0, 0), memory_space=pltpu.SMEM),
        ],
        out_shape=[
            jax.ShapeDtypeStruct((TOK_CHUNK,), jnp.int32),
            jax.ShapeDtypeStruct((1, 1), jnp.float32),
        ],
    )


def _gather_body(cb_hbm, idx_hbm, out_hbm, idx_v, rows_v, sem):
    wid = lax.axis_index("s") * SC_CORES + lax.axis_index("c")
    base = wid * B_PER_W
    pltpu.sync_copy(idx_hbm.at[pl.ds(base, B_PER_W)], idx_v)
    pltpu.async_copy(cb_hbm.at[idx_v], rows_v, sem).wait()
    pltpu.sync_copy(rows_v, out_hbm.at[pl.ds(base, B_PER_W), pl.ds(0, DIM)])


def _make_gather():
    return pl.kernel(
        _gather_body,
        mesh=plsc.VectorSubcoreMesh(core_axis_name="c", subcore_axis_name="s"),
        out_type=jax.ShapeDtypeStruct((TOK_CHUNK, PAD_DIM), jnp.float32),
        scratch_types=[
            pltpu.VMEM((B_PER_W,), jnp.int32),
            pltpu.VMEM((B_PER_W, DIM), jnp.float32),
            pltpu.SemaphoreType.DMA,
        ],
        compiler_params=pltpu.CompilerParams(use_tc_tiling_on_sc=False),
    )


TR_BLK = 4096


def _tr_body_first(g_ref, qt_ref):
    qt_ref[...] = g_ref[:, 0:DIM].T


def _tr_body_acc(g_ref, qt_in_ref, qt_ref):
    del qt_in_ref
    qt_ref[...] = g_ref[:, 0:DIM].T


def _make_transpose(chunk):
    blk_off = chunk * (TOK_CHUNK // TR_BLK)
    g_spec = pl.BlockSpec((TR_BLK, PAD_DIM), lambda i: (i, 0))
    out_spec = pl.BlockSpec((DIM, TR_BLK), lambda i: (0, i + blk_off))
    out_shape = jax.ShapeDtypeStruct((DIM, N_TOK), jnp.float32)
    if chunk == 0:
        return pl.pallas_call(
            _tr_body_first,
            grid=(TOK_CHUNK // TR_BLK,),
            in_specs=[g_spec],
            out_specs=out_spec,
            out_shape=out_shape,
        )
    return pl.pallas_call(
        _tr_body_acc,
        grid=(TOK_CHUNK // TR_BLK,),
        in_specs=[g_spec, pl.BlockSpec(memory_space=pl.ANY)],
        out_specs=out_spec,
        out_shape=out_shape,
        input_output_aliases={1: 0},
    )


def kernel(inputs, codebook):
    xt = inputs.T
    cbt = codebook.T
    idxs, gathers = [], []
    loss_sum = None
    for c in range(NCHUNK):
        idx_c, loss_c = _make_dist_argmin(c)(xt, cbt)
        g_c = _make_gather()(codebook, idx_c)
        idxs.append(idx_c)
        gathers.append(g_c)
        s = loss_c[0, 0]
        loss_sum = s if loss_sum is None else loss_sum + s
    qt = _make_transpose(0)(gathers[0])
    for c in range(1, NCHUNK):
        qt = _make_transpose(c)(gathers[c], qt)
    if NCHUNK == 1:
        idx = idxs[0]
    else:
        idx = jnp.zeros((N_TOK,), jnp.int32)
        for c in range(NCHUNK):
            idx = lax.dynamic_update_slice(idx, idxs[c], (c * TOK_CHUNK,))
    return qt.T, loss_sum * LOSS_SCALE, idx
